# Initial kernel scaffold; baseline (speedup 1.0000x reference)
import jax
import jax.numpy as jnp
from jax.experimental import pallas as pl


def kernel(h, edge_index, e, W, b, gamma, beta):
    raise NotImplementedError("write your pallas kernel here")


# speedup vs baseline: 3.4875x; 3.4875x over previous
import jax
import jax.numpy as jnp
from jax import lax
from jax.experimental import pallas as pl
from jax.experimental.pallas import tpu as pltpu
from jax.experimental.pallas import tpu_sc as plsc

NN = 10000
EE = 320000
DD = 128
EPSV = 1e-5

NC, NS = 2, 16
NW = NC * NS
EPW = EE // NW
EPT = EE // NS
HD = DD // 2
CHUNK = 80
NCHUNK = EPT // CHUNK
GROUPS = CHUNK // 16
STRIPE = 624
TAIL = NN - NS * STRIPE
LANES = 16

_SC_MESH = dict(core_axis_name="c", subcore_axis_name="s",
                num_cores=NC, num_subcores=NS)


def _deg_body(src_hbm, dst_hbm, outs_hbm, outd_hbm, src_v, dst_v, hs_v, hd_v):
    c = lax.axis_index("c")
    s = lax.axis_index("s")
    wid = s * NC + c
    pltpu.sync_copy(src_hbm.at[wid], src_v)
    pltpu.sync_copy(dst_hbm.at[wid], dst_v)
    zeros = jnp.zeros((LANES,), jnp.float32)

    def zbody(i, carry):
        hs_v[pl.ds(i * LANES, LANES)] = zeros
        hd_v[pl.ds(i * LANES, LANES)] = zeros
        return carry

    lax.fori_loop(0, NN // LANES, zbody, 0)
    ones = jnp.ones((LANES,), jnp.float32)

    def ebody(i, carry):
        si = src_v[pl.ds(i * LANES, LANES)]
        plsc.addupdate_scatter(hs_v, [si], ones)
        di = dst_v[pl.ds(i * LANES, LANES)]
        plsc.addupdate_scatter(hd_v, [di], ones)
        return carry

    lax.fori_loop(0, EPW // LANES, ebody, 0)
    pltpu.sync_copy(hs_v, outs_hbm.at[wid, 0])
    pltpu.sync_copy(hd_v, outd_hbm.at[wid, 0])


_deg_call = pl.kernel(
    _deg_body,
    out_type=(jax.ShapeDtypeStruct((NW, 1, NN), jnp.float32),
              jax.ShapeDtypeStruct((NW, 1, NN), jnp.float32)),
    mesh=plsc.VectorSubcoreMesh(**_SC_MESH),
    compiler_params=pltpu.CompilerParams(needs_layout_passes=False),
    scratch_types=[
        pltpu.VMEM((EPW,), jnp.int32),
        pltpu.VMEM((EPW,), jnp.int32),
        pltpu.VMEM((NN,), jnp.float32),
        pltpu.VMEM((NN,), jnp.float32),
    ],
)


def _feat_body(h_ref, hs_ref, feat_ref):
    deg = jnp.sum(hs_ref[...], axis=1, keepdims=True)
    ns = lax.rsqrt(jnp.maximum(deg, 1.0))
    feat = h_ref[...] * ns
    feat_ref[0] = feat[:, :HD]
    feat_ref[1] = feat[:, HD:]


_feat_call = pl.pallas_call(
    _feat_body,
    out_shape=jax.ShapeDtypeStruct((NC, NN, HD), jnp.float32),
)


def _agg_body(feat_hbm, src_hbm, dst_hbm, e_hbm, out_hbm,
              src_v, dst_v, e_v, rows_v, sem, agg_sh):
    c = lax.axis_index("c")
    s = lax.axis_index("s")
    pltpu.sync_copy(src_hbm.at[s], src_v)
    pltpu.sync_copy(dst_hbm.at[s], dst_v)
    pltpu.sync_copy(e_hbm.at[s], e_v)

    zeros = jnp.zeros((LANES,), jnp.float32)

    def zr(r, carry):
        for j in range(HD // LANES):
            rows_v[r, pl.ds(j * LANES, LANES)] = zeros
        return carry

    lax.fori_loop(0, CHUNK, zr, 0)
    base = s * STRIPE
    full, rem = divmod(STRIPE, CHUNK)
    for k in range(full):
        pltpu.sync_copy(rows_v, agg_sh.at[pl.ds(base + k * CHUNK, CHUNK)])
    if rem:
        pltpu.sync_copy(rows_v.at[pl.ds(0, rem)],
                        agg_sh.at[pl.ds(base + full * CHUNK, rem)])

    @pl.when(s == NS - 1)
    def _zero_tail():
        pltpu.sync_copy(rows_v.at[pl.ds(0, TAIL)],
                        agg_sh.at[pl.ds(NS * STRIPE, TAIL)])

    plsc.subcore_barrier()
    my_feat = feat_hbm.at[c]

    def chunk_body(i, carry):
        pltpu.async_copy(my_feat.at[src_v.at[i]], rows_v, sem).wait()

        def scale(g, carry2):
            ev = e_v[i, pl.ds(g * LANES, LANES)]
            for r in range(LANES):
                ce = ev[r]
                for j in range(HD // LANES):
                    sl = pl.ds(j * LANES, LANES)
                    rows_v[g * LANES + r, sl] = rows_v[g * LANES + r, sl] * ce
            return carry2

        lax.fori_loop(0, GROUPS, scale, 0)
        pltpu.sync_copy(rows_v, agg_sh.at[dst_v.at[i]], add=True)
        return carry

    lax.fori_loop(0, NCHUNK, chunk_body, 0)
    plsc.subcore_barrier()
    pltpu.sync_copy(agg_sh.at[pl.ds(base, STRIPE)],
                    out_hbm.at[c, pl.ds(base, STRIPE)])

    @pl.when(s == NS - 1)
    def _write_tail():
        pltpu.sync_copy(agg_sh.at[pl.ds(NS * STRIPE, TAIL)],
                        out_hbm.at[c, pl.ds(NS * STRIPE, TAIL)])


_agg_call = pl.kernel(
    _agg_body,
    out_type=jax.ShapeDtypeStruct((NC, NN, HD), jnp.float32),
    mesh=plsc.VectorSubcoreMesh(**_SC_MESH),
    compiler_params=pltpu.CompilerParams(needs_layout_passes=False,
                                         use_tc_tiling_on_sc=False),
    scratch_types=[
        pltpu.VMEM((NCHUNK, CHUNK), jnp.int32),
        pltpu.VMEM((NCHUNK, CHUNK), jnp.int32),
        pltpu.VMEM((NCHUNK, CHUNK), jnp.float32),
        pltpu.VMEM((CHUNK, HD), jnp.float32),
        pltpu.SemaphoreType.DMA,
        pltpu.VMEM_SHARED((NN, HD), jnp.float32),
    ],
)


def _out_body(p_ref, hd_ref, h_ref, w_ref, b_ref, g_ref, bt_ref, o_ref):
    deg = jnp.sum(hd_ref[...], axis=1, keepdims=True)
    nd = lax.rsqrt(jnp.maximum(deg, 1.0))
    agg = jnp.concatenate([p_ref[0], p_ref[1]], axis=-1) * nd
    out = jnp.dot(agg, w_ref[...], preferred_element_type=jnp.float32)
    out = out + b_ref[...]
    mean = jnp.mean(out, axis=0, keepdims=True)
    var = jnp.mean((out - mean) ** 2, axis=0, keepdims=True)
    out = (out - mean) * lax.rsqrt(var + EPSV) * g_ref[...] + bt_ref[...]
    o_ref[...] = jnp.maximum(out, 0.0) + h_ref[...]


_out_call = pl.pallas_call(
    _out_body,
    out_shape=jax.ShapeDtypeStruct((NN, DD), jnp.float32),
)


def kernel(h, edge_index, e, W, b, gamma, beta):
    src = edge_index[0]
    dst = edge_index[1]
    src2 = src.reshape(NW, EPW)
    dst2 = dst.reshape(NW, EPW)
    hs, hd = _deg_call(src2, dst2)
    hs_t = jnp.transpose(hs[:, 0, :])
    hd_t = jnp.transpose(hd[:, 0, :])
    feat2 = _feat_call(h, hs_t)
    src3 = src.reshape(NS, NCHUNK, CHUNK)
    dst3 = dst.reshape(NS, NCHUNK, CHUNK)
    e3 = e.reshape(NS, NCHUNK, CHUNK)
    partials = _agg_call(feat2, src3, dst3, e3)
    return _out_call(partials, hd_t, h, W,
                     b.reshape(1, DD), gamma.reshape(1, DD),
                     beta.reshape(1, DD))


# speedup vs baseline: 4.8640x; 1.3947x over previous
import jax
import jax.numpy as jnp
from jax import lax
from jax.experimental import pallas as pl
from jax.experimental.pallas import tpu as pltpu
from jax.experimental.pallas import tpu_sc as plsc

NN = 10000
EE = 320000
DD = 128
EPSV = 1e-5

NC, NS = 2, 16
NW = NC * NS
EPW = EE // NW
EPT = EE // NS
HD = DD // 2
CHUNK = 80
NCHUNK = EPT // CHUNK
GROUPS = CHUNK // 16
STRIPE = 624
TAIL = NN - NS * STRIPE
LANES = 16

_SC_MESH = dict(core_axis_name="c", subcore_axis_name="s",
                num_cores=NC, num_subcores=NS)


def _deg_body(src_hbm, dst_hbm, outs_hbm, outd_hbm, src_v, dst_v, hs_v, hd_v):
    c = lax.axis_index("c")
    s = lax.axis_index("s")
    wid = s * NC + c
    pltpu.sync_copy(src_hbm.at[wid], src_v)
    pltpu.sync_copy(dst_hbm.at[wid], dst_v)
    zeros = jnp.zeros((LANES,), jnp.float32)

    def zbody(i, carry):
        hs_v[pl.ds(i * LANES, LANES)] = zeros
        hd_v[pl.ds(i * LANES, LANES)] = zeros
        return carry

    lax.fori_loop(0, NN // LANES, zbody, 0)
    ones = jnp.ones((LANES,), jnp.float32)

    def ebody(i, carry):
        si = src_v[pl.ds(i * LANES, LANES)]
        plsc.addupdate_scatter(hs_v, [si], ones)
        di = dst_v[pl.ds(i * LANES, LANES)]
        plsc.addupdate_scatter(hd_v, [di], ones)
        return carry

    lax.fori_loop(0, EPW // LANES, ebody, 0)
    pltpu.sync_copy(hs_v, outs_hbm.at[wid, 0])
    pltpu.sync_copy(hd_v, outd_hbm.at[wid, 0])


_deg_call = pl.kernel(
    _deg_body,
    out_type=(jax.ShapeDtypeStruct((NW, 1, NN), jnp.float32),
              jax.ShapeDtypeStruct((NW, 1, NN), jnp.float32)),
    mesh=plsc.VectorSubcoreMesh(**_SC_MESH),
    compiler_params=pltpu.CompilerParams(needs_layout_passes=False),
    scratch_types=[
        pltpu.VMEM((EPW,), jnp.int32),
        pltpu.VMEM((EPW,), jnp.int32),
        pltpu.VMEM((NN,), jnp.float32),
        pltpu.VMEM((NN,), jnp.float32),
    ],
)


def _feat_body(h_ref, hs_ref, feat_ref):
    deg = jnp.sum(hs_ref[...], axis=1, keepdims=True)
    ns = lax.rsqrt(jnp.maximum(deg, 1.0))
    feat = h_ref[...] * ns
    feat_ref[0] = feat[:, :HD]
    feat_ref[1] = feat[:, HD:]


_feat_call = pl.pallas_call(
    _feat_body,
    out_shape=jax.ShapeDtypeStruct((NC, NN, HD), jnp.float32),
)


def _agg_body(feat_hbm, src_hbm, dst_hbm, e_hbm, out_hbm,
              src_v, dst_v, e_v, rows_a, rows_b, sem_a, sem_b, agg_sh):
    c = lax.axis_index("c")
    s = lax.axis_index("s")
    pltpu.sync_copy(src_hbm.at[s], src_v)
    pltpu.sync_copy(dst_hbm.at[s], dst_v)
    pltpu.sync_copy(e_hbm.at[s], e_v)

    zeros = jnp.zeros((LANES,), jnp.float32)

    def zr(r, carry):
        for j in range(HD // LANES):
            rows_a[r, pl.ds(j * LANES, LANES)] = zeros
        return carry

    lax.fori_loop(0, CHUNK, zr, 0)
    base = s * STRIPE
    full, rem = divmod(STRIPE, CHUNK)
    for k in range(full):
        pltpu.sync_copy(rows_a, agg_sh.at[pl.ds(base + k * CHUNK, CHUNK)])
    if rem:
        pltpu.sync_copy(rows_a.at[pl.ds(0, rem)],
                        agg_sh.at[pl.ds(base + full * CHUNK, rem)])

    @pl.when(s == NS - 1)
    def _zero_tail():
        pltpu.sync_copy(rows_a.at[pl.ds(0, TAIL)],
                        agg_sh.at[pl.ds(NS * STRIPE, TAIL)])

    plsc.subcore_barrier()
    my_feat = feat_hbm.at[c]

    def _scale(rows_v, i):
        def scale(g, carry2):
            ev = e_v[i, pl.ds(g * LANES, LANES)]
            for r in range(LANES):
                ce = ev[r]
                for j in range(HD // LANES):
                    sl = pl.ds(j * LANES, LANES)
                    rows_v[g * LANES + r, sl] = rows_v[g * LANES + r, sl] * ce
            return carry2

        lax.fori_loop(0, GROUPS, scale, 0)

    pltpu.async_copy(my_feat.at[src_v.at[0]], rows_a, sem_a)

    def chunk_body(i2, carry):
        i = i2 * 2
        pltpu.async_copy(my_feat.at[src_v.at[i + 1]], rows_b, sem_b)
        pltpu.make_async_copy(my_feat.at[src_v.at[i]], rows_a, sem_a).wait()
        _scale(rows_a, i)
        pltpu.sync_copy(rows_a, agg_sh.at[dst_v.at[i]], add=True)

        @pl.when(i2 < NCHUNK // 2 - 1)
        def _prefetch():
            pltpu.async_copy(my_feat.at[src_v.at[i + 2]], rows_a, sem_a)

        pltpu.make_async_copy(my_feat.at[src_v.at[i + 1]], rows_b, sem_b).wait()
        _scale(rows_b, i + 1)
        pltpu.sync_copy(rows_b, agg_sh.at[dst_v.at[i + 1]], add=True)
        return carry

    lax.fori_loop(0, NCHUNK // 2, chunk_body, 0)
    plsc.subcore_barrier()
    pltpu.sync_copy(agg_sh.at[pl.ds(base, STRIPE)],
                    out_hbm.at[c, pl.ds(base, STRIPE)])

    @pl.when(s == NS - 1)
    def _write_tail():
        pltpu.sync_copy(agg_sh.at[pl.ds(NS * STRIPE, TAIL)],
                        out_hbm.at[c, pl.ds(NS * STRIPE, TAIL)])


_agg_call = pl.kernel(
    _agg_body,
    out_type=jax.ShapeDtypeStruct((NC, NN, HD), jnp.float32),
    mesh=plsc.VectorSubcoreMesh(**_SC_MESH),
    compiler_params=pltpu.CompilerParams(needs_layout_passes=False,
                                         use_tc_tiling_on_sc=False),
    scratch_types=[
        pltpu.VMEM((NCHUNK, CHUNK), jnp.int32),
        pltpu.VMEM((NCHUNK, CHUNK), jnp.int32),
        pltpu.VMEM((NCHUNK, CHUNK), jnp.float32),
        pltpu.VMEM((CHUNK, HD), jnp.float32),
        pltpu.VMEM((CHUNK, HD), jnp.float32),
        pltpu.SemaphoreType.DMA,
        pltpu.SemaphoreType.DMA,
        pltpu.VMEM_SHARED((NN, HD), jnp.float32),
    ],
)


def _out_body(p_ref, hd_ref, h_ref, w_ref, b_ref, g_ref, bt_ref, o_ref):
    deg = jnp.sum(hd_ref[...], axis=1, keepdims=True)
    nd = lax.rsqrt(jnp.maximum(deg, 1.0))
    agg = jnp.concatenate([p_ref[0], p_ref[1]], axis=-1) * nd
    out = jnp.dot(agg, w_ref[...], preferred_element_type=jnp.float32)
    out = out + b_ref[...]
    mean = jnp.mean(out, axis=0, keepdims=True)
    var = jnp.mean((out - mean) ** 2, axis=0, keepdims=True)
    out = (out - mean) * lax.rsqrt(var + EPSV) * g_ref[...] + bt_ref[...]
    o_ref[...] = jnp.maximum(out, 0.0) + h_ref[...]


_out_call = pl.pallas_call(
    _out_body,
    out_shape=jax.ShapeDtypeStruct((NN, DD), jnp.float32),
)


def kernel(h, edge_index, e, W, b, gamma, beta):
    src = edge_index[0]
    dst = edge_index[1]
    src2 = src.reshape(NW, EPW)
    dst2 = dst.reshape(NW, EPW)
    hs, hd = _deg_call(src2, dst2)
    hs_t = jnp.transpose(hs[:, 0, :])
    hd_t = jnp.transpose(hd[:, 0, :])
    feat2 = _feat_call(h, hs_t)
    src3 = src.reshape(NS, NCHUNK, CHUNK)
    dst3 = dst.reshape(NS, NCHUNK, CHUNK)
    e3 = e.reshape(NS, NCHUNK, CHUNK)
    partials = _agg_call(feat2, src3, dst3, e3)
    return _out_call(partials, hd_t, h, W,
                     b.reshape(1, DD), gamma.reshape(1, DD),
                     beta.reshape(1, DD))


# speedup vs baseline: 5.5153x; 1.1339x over previous
import jax
import jax.numpy as jnp
from jax import lax
from jax.experimental import pallas as pl
from jax.experimental.pallas import tpu as pltpu
from jax.experimental.pallas import tpu_sc as plsc

NN = 10000
EE = 320000
DD = 128
EPSV = 1e-5

NC, NS = 2, 16
NW = NC * NS
EPW = EE // NW
EPT = EE // NS
HD = DD // 2
CHUNK = 80
NCHUNK = EPT // CHUNK
GROUPS = CHUNK // 16
STRIPE = 624
TAIL = NN - NS * STRIPE
LANES = 16

_SC_MESH = dict(core_axis_name="c", subcore_axis_name="s",
                num_cores=NC, num_subcores=NS)


def _deg_body(src_hbm, dst_hbm, outs_hbm, outd_hbm, src_v, dst_v, hs_v, hd_v):
    c = lax.axis_index("c")
    s = lax.axis_index("s")
    wid = s * NC + c
    pltpu.sync_copy(src_hbm.at[wid], src_v)
    pltpu.sync_copy(dst_hbm.at[wid], dst_v)
    zeros = jnp.zeros((LANES,), jnp.float32)

    def zbody(i, carry):
        hs_v[pl.ds(i * LANES, LANES)] = zeros
        hd_v[pl.ds(i * LANES, LANES)] = zeros
        return carry

    lax.fori_loop(0, NN // LANES, zbody, 0)
    ones = jnp.ones((LANES,), jnp.float32)

    def ebody(i, carry):
        si = src_v[pl.ds(i * LANES, LANES)]
        plsc.addupdate_scatter(hs_v, [si], ones)
        di = dst_v[pl.ds(i * LANES, LANES)]
        plsc.addupdate_scatter(hd_v, [di], ones)
        return carry

    lax.fori_loop(0, EPW // LANES, ebody, 0)
    pltpu.sync_copy(hs_v, outs_hbm.at[wid, 0])
    pltpu.sync_copy(hd_v, outd_hbm.at[wid, 0])


_deg_call = pl.kernel(
    _deg_body,
    out_type=(jax.ShapeDtypeStruct((NW, 1, NN), jnp.float32),
              jax.ShapeDtypeStruct((NW, 1, NN), jnp.float32)),
    mesh=plsc.VectorSubcoreMesh(**_SC_MESH),
    compiler_params=pltpu.CompilerParams(needs_layout_passes=False),
    scratch_types=[
        pltpu.VMEM((EPW,), jnp.int32),
        pltpu.VMEM((EPW,), jnp.int32),
        pltpu.VMEM((NN,), jnp.float32),
        pltpu.VMEM((NN,), jnp.float32),
    ],
)


def _feat_body(h_ref, hs_ref, feat_ref):
    deg = jnp.sum(hs_ref[...], axis=1, keepdims=True)
    ns = lax.rsqrt(jnp.maximum(deg, 1.0))
    feat = h_ref[...] * ns
    feat_ref[0] = feat[:, :HD]
    feat_ref[1] = feat[:, HD:]


_feat_call = pl.pallas_call(
    _feat_body,
    out_shape=jax.ShapeDtypeStruct((NC, NN, HD), jnp.float32),
)


def _agg_body(feat_hbm, src_hbm, dst_hbm, e_hbm, out_hbm,
              src_v, dst_v, e_v, rows_a, rows_b, rows_c, rows_d,
              gs_a, gs_b, gs_c, gs_d, ss_a, ss_b, ss_c, ss_d, agg_sh):
    c = lax.axis_index("c")
    s = lax.axis_index("s")
    pltpu.sync_copy(src_hbm.at[s], src_v)
    pltpu.sync_copy(dst_hbm.at[s], dst_v)
    pltpu.sync_copy(e_hbm.at[s], e_v)

    zeros = jnp.zeros((LANES,), jnp.float32)

    def zr(r, carry):
        for j in range(HD // LANES):
            rows_a[r, pl.ds(j * LANES, LANES)] = zeros
        return carry

    lax.fori_loop(0, CHUNK, zr, 0)
    base = s * STRIPE
    full, rem = divmod(STRIPE, CHUNK)
    for k in range(full):
        pltpu.sync_copy(rows_a, agg_sh.at[pl.ds(base + k * CHUNK, CHUNK)])
    if rem:
        pltpu.sync_copy(rows_a.at[pl.ds(0, rem)],
                        agg_sh.at[pl.ds(base + full * CHUNK, rem)])

    @pl.when(s == NS - 1)
    def _zero_tail():
        pltpu.sync_copy(rows_a.at[pl.ds(0, TAIL)],
                        agg_sh.at[pl.ds(NS * STRIPE, TAIL)])

    plsc.subcore_barrier()
    my_feat = feat_hbm.at[c]

    def _scale(rows_v, i):
        def scale(g, carry2):
            ev = e_v[i, pl.ds(g * LANES, LANES)]
            for r in range(LANES):
                ce = ev[r]
                for j in range(HD // LANES):
                    sl = pl.ds(j * LANES, LANES)
                    rows_v[g * LANES + r, sl] = rows_v[g * LANES + r, sl] * ce
            return carry2

        lax.fori_loop(0, GROUPS, scale, 0)

    rows = (rows_a, rows_b, rows_c, rows_d)
    gs = (gs_a, gs_b, gs_c, gs_d)
    ss = (ss_a, ss_b, ss_c, ss_d)

    def _visit(ic, k, guard_lo):
        kf = (k + 2) % 4
        if guard_lo is None:
            pltpu.make_async_copy(rows[kf], agg_sh.at[dst_v.at[ic - 2]],
                                  ss[kf]).wait()
        else:
            @pl.when(guard_lo)
            def _drain():
                pltpu.make_async_copy(rows[kf], agg_sh.at[dst_v.at[ic - 2]],
                                      ss[kf]).wait()

        if not isinstance(ic, int) or ic + 2 < NCHUNK:
            pltpu.async_copy(my_feat.at[src_v.at[ic + 2]], rows[kf], gs[kf])
        pltpu.make_async_copy(my_feat.at[src_v.at[ic]], rows[k], gs[k]).wait()
        _scale(rows[k], ic)
        pltpu.async_copy(rows[k], agg_sh.at[dst_v.at[ic]], ss[k], add=True)

    pltpu.async_copy(my_feat.at[src_v.at[0]], rows_a, gs_a)
    pltpu.async_copy(my_feat.at[src_v.at[1]], rows_b, gs_b)

    def chunk_body(i2, carry):
        i = i2 * 4
        _visit(i + 0, 0, i2 > 0)
        _visit(i + 1, 1, i2 > 0)
        _visit(i + 2, 2, None)
        _visit(i + 3, 3, None)
        return carry

    lax.fori_loop(0, NCHUNK // 4, chunk_body, 0)
    for ic in range(NCHUNK - NCHUNK % 4, NCHUNK):
        _visit(ic, ic % 4, None)
    for ic in range(NCHUNK - 2, NCHUNK):
        k = ic % 4
        pltpu.make_async_copy(rows[k], agg_sh.at[dst_v.at[ic]], ss[k]).wait()
    plsc.subcore_barrier()
    pltpu.sync_copy(agg_sh.at[pl.ds(base, STRIPE)],
                    out_hbm.at[c, pl.ds(base, STRIPE)])

    @pl.when(s == NS - 1)
    def _write_tail():
        pltpu.sync_copy(agg_sh.at[pl.ds(NS * STRIPE, TAIL)],
                        out_hbm.at[c, pl.ds(NS * STRIPE, TAIL)])


_agg_call = pl.kernel(
    _agg_body,
    out_type=jax.ShapeDtypeStruct((NC, NN, HD), jnp.float32),
    mesh=plsc.VectorSubcoreMesh(**_SC_MESH),
    compiler_params=pltpu.CompilerParams(needs_layout_passes=False,
                                         use_tc_tiling_on_sc=False),
    scratch_types=[
        pltpu.VMEM((NCHUNK, CHUNK), jnp.int32),
        pltpu.VMEM((NCHUNK, CHUNK), jnp.int32),
        pltpu.VMEM((NCHUNK, CHUNK), jnp.float32),
        pltpu.VMEM((CHUNK, HD), jnp.float32),
        pltpu.VMEM((CHUNK, HD), jnp.float32),
        pltpu.VMEM((CHUNK, HD), jnp.float32),
        pltpu.VMEM((CHUNK, HD), jnp.float32),
        pltpu.SemaphoreType.DMA,
        pltpu.SemaphoreType.DMA,
        pltpu.SemaphoreType.DMA,
        pltpu.SemaphoreType.DMA,
        pltpu.SemaphoreType.DMA,
        pltpu.SemaphoreType.DMA,
        pltpu.SemaphoreType.DMA,
        pltpu.SemaphoreType.DMA,
        pltpu.VMEM_SHARED((NN, HD), jnp.float32),
    ],
)


def _out_body(p_ref, hd_ref, h_ref, w_ref, b_ref, g_ref, bt_ref, o_ref):
    deg = jnp.sum(hd_ref[...], axis=1, keepdims=True)
    nd = lax.rsqrt(jnp.maximum(deg, 1.0))
    agg = jnp.concatenate([p_ref[0], p_ref[1]], axis=-1) * nd
    out = jnp.dot(agg, w_ref[...], preferred_element_type=jnp.float32)
    out = out + b_ref[...]
    mean = jnp.mean(out, axis=0, keepdims=True)
    var = jnp.mean((out - mean) ** 2, axis=0, keepdims=True)
    out = (out - mean) * lax.rsqrt(var + EPSV) * g_ref[...] + bt_ref[...]
    o_ref[...] = jnp.maximum(out, 0.0) + h_ref[...]


_out_call = pl.pallas_call(
    _out_body,
    out_shape=jax.ShapeDtypeStruct((NN, DD), jnp.float32),
)


def kernel(h, edge_index, e, W, b, gamma, beta):
    src = edge_index[0]
    dst = edge_index[1]
    src2 = src.reshape(NW, EPW)
    dst2 = dst.reshape(NW, EPW)
    hs, hd = _deg_call(src2, dst2)
    hs_t = jnp.transpose(hs[:, 0, :])
    hd_t = jnp.transpose(hd[:, 0, :])
    feat2 = _feat_call(h, hs_t)
    src3 = src.reshape(NS, NCHUNK, CHUNK)
    dst3 = dst.reshape(NS, NCHUNK, CHUNK)
    e3 = e.reshape(NS, NCHUNK, CHUNK)
    partials = _agg_call(feat2, src3, dst3, e3)
    return _out_call(partials, hd_t, h, W,
                     b.reshape(1, DD), gamma.reshape(1, DD),
                     beta.reshape(1, DD))


# speedup vs baseline: 5.5230x; 1.0014x over previous
import jax
import jax.numpy as jnp
from jax import lax
from jax.experimental import pallas as pl
from jax.experimental.pallas import tpu as pltpu
from jax.experimental.pallas import tpu_sc as plsc

NN = 10000
EE = 320000
DD = 128
EPSV = 1e-5

NC, NS = 2, 16
NW = NC * NS
EPW = EE // NW
EPT = EE // NS
HD = DD // 2
CHUNK = 80
NCHUNK = EPT // CHUNK
GROUPS = CHUNK // 16
STRIPE = 624
TAIL = NN - NS * STRIPE
LANES = 16

_SC_MESH = dict(core_axis_name="c", subcore_axis_name="s",
                num_cores=NC, num_subcores=NS)


def _deg_body(src_hbm, dst_hbm, outs_hbm, outd_hbm, src_v, dst_v, hs_v, hd_v):
    c = lax.axis_index("c")
    s = lax.axis_index("s")
    wid = s * NC + c
    pltpu.sync_copy(src_hbm.at[wid], src_v)
    pltpu.sync_copy(dst_hbm.at[wid], dst_v)
    zeros = jnp.zeros((LANES,), jnp.float32)

    def zbody(i, carry):
        hs_v[pl.ds(i * LANES, LANES)] = zeros
        hd_v[pl.ds(i * LANES, LANES)] = zeros
        return carry

    lax.fori_loop(0, NN // LANES, zbody, 0)
    ones = jnp.ones((LANES,), jnp.float32)

    def ebody(i, carry):
        si = src_v[pl.ds(i * LANES, LANES)]
        plsc.addupdate_scatter(hs_v, [si], ones)
        di = dst_v[pl.ds(i * LANES, LANES)]
        plsc.addupdate_scatter(hd_v, [di], ones)
        return carry

    lax.fori_loop(0, EPW // LANES, ebody, 0)
    pltpu.sync_copy(hs_v, outs_hbm.at[wid, 0])
    pltpu.sync_copy(hd_v, outd_hbm.at[wid, 0])


_deg_call = pl.kernel(
    _deg_body,
    out_type=(jax.ShapeDtypeStruct((NW, 1, NN), jnp.float32),
              jax.ShapeDtypeStruct((NW, 1, NN), jnp.float32)),
    mesh=plsc.VectorSubcoreMesh(**_SC_MESH),
    compiler_params=pltpu.CompilerParams(needs_layout_passes=False),
    scratch_types=[
        pltpu.VMEM((EPW,), jnp.int32),
        pltpu.VMEM((EPW,), jnp.int32),
        pltpu.VMEM((NN,), jnp.float32),
        pltpu.VMEM((NN,), jnp.float32),
    ],
)


def _feat_body(h_ref, hs_ref, feat_ref):
    deg = jnp.sum(hs_ref[...], axis=1, keepdims=True)
    ns = lax.rsqrt(jnp.maximum(deg, 1.0))
    feat = h_ref[...] * ns
    feat_ref[0] = feat[:, :HD]
    feat_ref[1] = feat[:, HD:]


_feat_call = pl.pallas_call(
    _feat_body,
    out_shape=jax.ShapeDtypeStruct((NC, NN, HD), jnp.float32),
)


def _agg_body(feat_hbm, src_hbm, dst_hbm, e_hbm, out_hbm,
              src_v, dst_v, e_v, rows_a, rows_b, rows_c, rows_d,
              gs_a, gs_b, gs_c, gs_d, ss_a, ss_b, ss_c, ss_d, agg_sh):
    c = lax.axis_index("c")
    s = lax.axis_index("s")
    pltpu.sync_copy(src_hbm.at[s], src_v)
    pltpu.sync_copy(dst_hbm.at[s], dst_v)
    pltpu.sync_copy(e_hbm.at[s], e_v)

    zeros = jnp.zeros((LANES,), jnp.float32)

    def zr(r, carry):
        for j in range(HD // LANES):
            rows_a[r, pl.ds(j * LANES, LANES)] = zeros
        return carry

    lax.fori_loop(0, CHUNK, zr, 0)
    base = s * STRIPE
    full, rem = divmod(STRIPE, CHUNK)
    for k in range(full):
        pltpu.sync_copy(rows_a, agg_sh.at[pl.ds(base + k * CHUNK, CHUNK)])
    if rem:
        pltpu.sync_copy(rows_a.at[pl.ds(0, rem)],
                        agg_sh.at[pl.ds(base + full * CHUNK, rem)])

    @pl.when(s == NS - 1)
    def _zero_tail():
        pltpu.sync_copy(rows_a.at[pl.ds(0, TAIL)],
                        agg_sh.at[pl.ds(NS * STRIPE, TAIL)])

    plsc.subcore_barrier()
    my_feat = feat_hbm.at[c]

    def _scale(rows_v, i):
        def scale(g, carry2):
            ev = e_v[i, pl.ds(g * LANES, LANES)]
            for r in range(LANES):
                cev = lax.gather(
                    ev, jnp.full((LANES, 1), r, jnp.int32),
                    lax.GatherDimensionNumbers(offset_dims=(),
                                               collapsed_slice_dims=(0,),
                                               start_index_map=(0,)),
                    (1,), mode=lax.GatherScatterMode.PROMISE_IN_BOUNDS)
                for j in range(HD // LANES):
                    sl = pl.ds(j * LANES, LANES)
                    rows_v[g * LANES + r, sl] = rows_v[g * LANES + r, sl] * cev
            return carry2

        lax.fori_loop(0, GROUPS, scale, 0)

    rows = (rows_a, rows_b, rows_c, rows_d)
    gs = (gs_a, gs_b, gs_c, gs_d)
    ss = (ss_a, ss_b, ss_c, ss_d)

    def _visit(ic, k, guard_lo):
        kf = (k + 2) % 4
        if guard_lo is None:
            pltpu.make_async_copy(rows[kf], agg_sh.at[dst_v.at[ic - 2]],
                                  ss[kf]).wait()
        else:
            @pl.when(guard_lo)
            def _drain():
                pltpu.make_async_copy(rows[kf], agg_sh.at[dst_v.at[ic - 2]],
                                      ss[kf]).wait()

        if not isinstance(ic, int) or ic + 2 < NCHUNK:
            pltpu.async_copy(my_feat.at[src_v.at[ic + 2]], rows[kf], gs[kf])
        pltpu.make_async_copy(my_feat.at[src_v.at[ic]], rows[k], gs[k]).wait()
        _scale(rows[k], ic)
        pltpu.async_copy(rows[k], agg_sh.at[dst_v.at[ic]], ss[k], add=True)

    pltpu.async_copy(my_feat.at[src_v.at[0]], rows_a, gs_a)
    pltpu.async_copy(my_feat.at[src_v.at[1]], rows_b, gs_b)

    def chunk_body(i2, carry):
        i = i2 * 4
        _visit(i + 0, 0, i2 > 0)
        _visit(i + 1, 1, i2 > 0)
        _visit(i + 2, 2, None)
        _visit(i + 3, 3, None)
        return carry

    lax.fori_loop(0, NCHUNK // 4, chunk_body, 0)
    for ic in range(NCHUNK - NCHUNK % 4, NCHUNK):
        _visit(ic, ic % 4, None)
    for ic in range(NCHUNK - 2, NCHUNK):
        k = ic % 4
        pltpu.make_async_copy(rows[k], agg_sh.at[dst_v.at[ic]], ss[k]).wait()
    plsc.subcore_barrier()
    pltpu.sync_copy(agg_sh.at[pl.ds(base, STRIPE)],
                    out_hbm.at[c, pl.ds(base, STRIPE)])

    @pl.when(s == NS - 1)
    def _write_tail():
        pltpu.sync_copy(agg_sh.at[pl.ds(NS * STRIPE, TAIL)],
                        out_hbm.at[c, pl.ds(NS * STRIPE, TAIL)])


_agg_call = pl.kernel(
    _agg_body,
    out_type=jax.ShapeDtypeStruct((NC, NN, HD), jnp.float32),
    mesh=plsc.VectorSubcoreMesh(**_SC_MESH),
    compiler_params=pltpu.CompilerParams(needs_layout_passes=False,
                                         use_tc_tiling_on_sc=False),
    scratch_types=[
        pltpu.VMEM((NCHUNK, CHUNK), jnp.int32),
        pltpu.VMEM((NCHUNK, CHUNK), jnp.int32),
        pltpu.VMEM((NCHUNK, CHUNK), jnp.float32),
        pltpu.VMEM((CHUNK, HD), jnp.float32),
        pltpu.VMEM((CHUNK, HD), jnp.float32),
        pltpu.VMEM((CHUNK, HD), jnp.float32),
        pltpu.VMEM((CHUNK, HD), jnp.float32),
        pltpu.SemaphoreType.DMA,
        pltpu.SemaphoreType.DMA,
        pltpu.SemaphoreType.DMA,
        pltpu.SemaphoreType.DMA,
        pltpu.SemaphoreType.DMA,
        pltpu.SemaphoreType.DMA,
        pltpu.SemaphoreType.DMA,
        pltpu.SemaphoreType.DMA,
        pltpu.VMEM_SHARED((NN, HD), jnp.float32),
    ],
)


def _out_body(p_ref, hd_ref, h_ref, w_ref, b_ref, g_ref, bt_ref, o_ref):
    deg = jnp.sum(hd_ref[...], axis=1, keepdims=True)
    nd = lax.rsqrt(jnp.maximum(deg, 1.0))
    agg = jnp.concatenate([p_ref[0], p_ref[1]], axis=-1) * nd
    out = jnp.dot(agg, w_ref[...], preferred_element_type=jnp.float32)
    out = out + b_ref[...]
    mean = jnp.mean(out, axis=0, keepdims=True)
    var = jnp.mean((out - mean) ** 2, axis=0, keepdims=True)
    out = (out - mean) * lax.rsqrt(var + EPSV) * g_ref[...] + bt_ref[...]
    o_ref[...] = jnp.maximum(out, 0.0) + h_ref[...]


_out_call = pl.pallas_call(
    _out_body,
    out_shape=jax.ShapeDtypeStruct((NN, DD), jnp.float32),
)


def kernel(h, edge_index, e, W, b, gamma, beta):
    src = edge_index[0]
    dst = edge_index[1]
    src2 = src.reshape(NW, EPW)
    dst2 = dst.reshape(NW, EPW)
    hs, hd = _deg_call(src2, dst2)
    hs_t = jnp.transpose(hs[:, 0, :])
    hd_t = jnp.transpose(hd[:, 0, :])
    feat2 = _feat_call(h, hs_t)
    src3 = src.reshape(NS, NCHUNK, CHUNK)
    dst3 = dst.reshape(NS, NCHUNK, CHUNK)
    e3 = e.reshape(NS, NCHUNK, CHUNK)
    partials = _agg_call(feat2, src3, dst3, e3)
    return _out_call(partials, hd_t, h, W,
                     b.reshape(1, DD), gamma.reshape(1, DD),
                     beta.reshape(1, DD))
